# baseline (speedup 1.0000x reference)
import functools

import jax
import jax.numpy as jnp
from jax import lax
from jax.experimental import pallas as pl
from jax.experimental.pallas import tpu as pltpu
from jax.experimental.pallas import tpu_sc as plsc

B, H, N = 32, 8, 100000
L = 16
NC, NS = 2, 16
TB = 128
CS = 61440
CTW = 20
WC = CTW * TB
NCH = (CS // TB) // CTW
RING = 6
WB = 4096
JOFF = CS // WB
NB = (N - CS + WB - 1) // WB
BIG = 2**30


@functools.partial(
    pl.kernel,
    mesh=plsc.VectorSubcoreMesh(core_axis_name="c", subcore_axis_name="s"),
    out_type=[
        jax.ShapeDtypeStruct((B * L,), jnp.float32),
        jax.ShapeDtypeStruct((B * L,), jnp.int32),
    ],
    scratch_types=[
        pltpu.VMEM((RING, H, WC), jnp.float32),
        pltpu.VMEM((L,), jnp.float32),
        pltpu.VMEM((L,), jnp.int32),
        pltpu.SemaphoreType.DMA,
        pltpu.SemaphoreType.DMA,
        pltpu.SemaphoreType.DMA,
        pltpu.SemaphoreType.DMA,
        pltpu.SemaphoreType.DMA,
        pltpu.SemaphoreType.DMA,
    ],
)
def _argmax_sc(x_hbm, outv_hbm, outi_hbm, buf, resm, resi,
               sem0, sem1, sem2, sem3, sem4, sem5):
    c = lax.axis_index("c")
    s = lax.axis_index("s")
    b = s * NC + c
    sems = (sem0, sem1, sem2, sem3, sem4, sem5)
    iota = lax.iota(jnp.int32, L)

    def start(ci):
        slot = ci % RING
        return pltpu.async_copy(
            x_hbm.at[b, :, pl.ds(ci * WC, WC)],
            buf.at[slot],
            sems[slot])

    cps = [start(ci) for ci in range(RING)]

    ms = [jnp.full((L,), -jnp.inf, dtype=jnp.float32) for _ in range(H)]
    aas = [jnp.zeros((L,), dtype=jnp.int32) for _ in range(H)]

    for ci in range(NCH):
        slot = ci % RING
        cps[slot].wait()
        gbase = (ci * WC) // L

        def body(g, carry, slot=slot, gbase=gbase):
            mm = list(carry[:H])
            aa = list(carry[H:])
            col = g * L
            gv = jnp.broadcast_to(gbase + g, (L,))
            for r in range(H):
                v = buf[slot, r, pl.ds(col, L)]
                gt = v > mm[r]
                mm[r] = jnp.where(gt, v, mm[r])
                aa[r] = jnp.where(gt, gv, aa[r])
            return tuple(mm) + tuple(aa)

        carry = lax.fori_loop(0, WC // L, body, tuple(ms) + tuple(aas))
        ms, aas = list(carry[:H]), list(carry[H:])
        if ci + RING < NCH:
            cps[slot] = start(ci + RING)

    rv = jnp.zeros((L,), dtype=jnp.float32)
    ri = jnp.zeros((L,), dtype=jnp.int32)
    for r in range(H):
        rm = ms[r]
        ra = (aas[r] << 4) + iota
        for sh in (8, 4, 2, 1):
            perm = iota ^ sh
            mo = rm.at[perm].get(mode="promise_in_bounds")
            ao = ra.at[perm].get(mode="promise_in_bounds")
            better = (mo > rm) | ((mo == rm) & (ao < ra))
            rm = jnp.where(better, mo, rm)
            ra = jnp.where(better, ao, ra)
        rv = jnp.where(iota == r, rm, rv)
        ri = jnp.where(iota == r, ra, ri)

    resm[...] = rv
    resi[...] = ri
    off = pl.multiple_of(b * L, 8)
    pltpu.sync_copy(resm, outv_hbm.at[pl.ds(off, L)])
    pltpu.sync_copy(resi, outi_hbm.at[pl.ds(off, L)])


BBLK = 8


def _tc_body(x_ref, vout_ref, iout_ref, mscr, iscr):
    j = pl.program_id(1)
    iota2 = lax.broadcasted_iota(jnp.int32, (BBLK, H, WB), 2) + (j + JOFF) * WB
    v = jnp.where(iota2 < N, x_ref[...], -jnp.inf)
    rowmax = jnp.max(v, axis=2)
    cand = jnp.where(v == rowmax[:, :, None], iota2, BIG)
    rowidx = jnp.min(cand, axis=2)

    @pl.when(j == 0)
    def _():
        mscr[...] = rowmax
        iscr[...] = rowidx

    @pl.when(j > 0)
    def _():
        gt = rowmax > mscr[...]
        mscr[...] = jnp.where(gt, rowmax, mscr[...])
        iscr[...] = jnp.where(gt, rowidx, iscr[...])

    vout_ref[...] = mscr[...]
    iout_ref[...] = iscr[...]


_argmax_tc = pl.pallas_call(
    _tc_body,
    grid=(B // BBLK, NB),
    in_specs=[pl.BlockSpec((BBLK, H, WB), lambda i, j: (i, 0, j + JOFF))],
    out_specs=[
        pl.BlockSpec((BBLK, H), lambda i, j: (i, 0)),
        pl.BlockSpec((BBLK, H), lambda i, j: (i, 0)),
    ],
    out_shape=[
        jax.ShapeDtypeStruct((B, H), jnp.float32),
        jax.ShapeDtypeStruct((B, H), jnp.int32),
    ],
    scratch_shapes=[
        pltpu.VMEM((BBLK, H), jnp.float32),
        pltpu.VMEM((BBLK, H), jnp.int32),
    ],
    compiler_params=pltpu.CompilerParams(
        dimension_semantics=("parallel", "arbitrary")),
)


def _merge_body(scv_ref, sci_ref, tcv_ref, tci_ref, out_ref):
    scv, sci = scv_ref[...], sci_ref[...]
    tcv, tci = tcv_ref[...], tci_ref[...]
    take_tc = tcv > scv
    idx = jnp.where(take_tc, tci, sci)
    out_ref[...] = idx.astype(jnp.float32) / jnp.float32(N)


_merge = pl.pallas_call(
    _merge_body,
    out_shape=jax.ShapeDtypeStruct((B, H), jnp.float32),
)


def kernel(x):
    scv, sci = _argmax_sc(x)
    tcv, tci = _argmax_tc(x)
    scv2 = scv.reshape(B, L)[:, :H]
    sci2 = sci.reshape(B, L)[:, :H]
    return _merge(scv2, sci2, tcv, tci)


# speedup vs baseline: 1.1685x; 1.1685x over previous
import functools

import jax
import jax.numpy as jnp
from jax import lax
from jax.experimental import pallas as pl
from jax.experimental.pallas import tpu as pltpu
from jax.experimental.pallas import tpu_sc as plsc

B, H, N = 32, 8, 100000
L = 16
NC, NS = 2, 16
TB = 128
NT = N // TB
CTW = 20
WC = CTW * TB
RING = 6
NROUND = 6
NMAIN = NROUND * RING
NCH = NT // CTW
REMT = NT - NCH * CTW
TAILC = N - NT * TB
GREM = (NCH * CTW * TB) // L
GTAIL = (NT * TB) // L


@functools.partial(
    pl.kernel,
    mesh=plsc.VectorSubcoreMesh(core_axis_name="c", subcore_axis_name="s"),
    out_type=jax.ShapeDtypeStruct((B * L,), jnp.float32),
    scratch_types=[
        pltpu.VMEM((RING, H, WC), jnp.float32),
        pltpu.VMEM((H, TB), jnp.float32),
        pltpu.VMEM((H, TB), jnp.float32),
        pltpu.VMEM((L,), jnp.float32),
        pltpu.SemaphoreType.DMA,
        pltpu.SemaphoreType.DMA,
        pltpu.SemaphoreType.DMA,
        pltpu.SemaphoreType.DMA,
        pltpu.SemaphoreType.DMA,
        pltpu.SemaphoreType.DMA,
        pltpu.SemaphoreType.DMA,
        pltpu.SemaphoreType.DMA,
    ],
)
def _argmax_sc(x_hbm, xt_hbm, out_hbm, buf, rembuf, tbuf, res,
               sem0, sem1, sem2, sem3, sem4, sem5, semr, semt):
    c = lax.axis_index("c")
    s = lax.axis_index("s")
    b = s * NC + c
    sems = (sem0, sem1, sem2, sem3, sem4, sem5)
    iota = lax.iota(jnp.int32, L)

    def chunk_copy(ci, slot):
        return pltpu.make_async_copy(
            x_hbm.at[b, :, pl.ds(ci * WC, WC)], buf.at[slot], sems[slot])

    for k in range(RING):
        chunk_copy(k, k).start()
    pltpu.make_async_copy(
        x_hbm.at[b, :, pl.ds(NCH * WC, REMT * TB)],
        rembuf, semr).start()
    pltpu.make_async_copy(xt_hbm.at[b], tbuf, semt).start()

    def scan_groups(bufref, gbase, ngroups, ms, aas):
        def body(g, carry):
            mm = list(carry[:H])
            aa = list(carry[H:])
            col = g * L
            gv = jnp.broadcast_to(gbase + g, (L,))
            for r in range(H):
                v = bufref[r, pl.ds(col, L)]
                gt = v > mm[r]
                mm[r] = jnp.where(gt, v, mm[r])
                aa[r] = jnp.where(gt, gv, aa[r])
            return tuple(mm) + tuple(aa)

        carry = lax.fori_loop(0, ngroups, body, tuple(ms) + tuple(aas))
        return list(carry[:H]), list(carry[H:])

    ms = [jnp.full((L,), -jnp.inf, dtype=jnp.float32) for _ in range(H)]
    aas = [jnp.zeros((L,), dtype=jnp.int32) for _ in range(H)]

    def round_body(t, carry):
        ms = list(carry[:H])
        aas = list(carry[H:])
        for k in range(RING):
            ci = t * RING + k
            chunk_copy(ci, k).wait()
            ms, aas = scan_groups(buf.at[k], ci * (WC // L), WC // L, ms, aas)

            @pl.when(ci + RING < NCH)
            def _(ci=ci, k=k):
                chunk_copy(ci + RING, k).start()

        return tuple(ms) + tuple(aas)

    carry = lax.fori_loop(0, NROUND, round_body,
                          tuple(ms) + tuple(aas))
    ms, aas = list(carry[:H]), list(carry[H:])

    for ci in range(NMAIN, NCH):
        k = ci % RING
        chunk_copy(ci, k).wait()
        ms, aas = scan_groups(buf.at[k], ci * (WC // L), WC // L, ms, aas)

    pltpu.make_async_copy(
        x_hbm.at[b, :, pl.ds(NCH * WC, REMT * TB)], rembuf, semr).wait()
    ms, aas = scan_groups(rembuf, GREM, (REMT * TB) // L, ms, aas)
    pltpu.make_async_copy(xt_hbm.at[b], tbuf, semt).wait()
    ms, aas = scan_groups(tbuf, GTAIL, TB // L, ms, aas)

    resv = jnp.zeros((L,), dtype=jnp.float32)
    for r in range(H):
        rm = ms[r]
        ra = (aas[r] << 4) + iota
        for sh in (8, 4, 2, 1):
            perm = iota ^ sh
            mo = rm.at[perm].get(mode="promise_in_bounds")
            ao = ra.at[perm].get(mode="promise_in_bounds")
            better = (mo > rm) | ((mo == rm) & (ao < ra))
            rm = jnp.where(better, mo, rm)
            ra = jnp.where(better, ao, ra)
        val = ra.astype(jnp.float32) / jnp.float32(N)
        resv = jnp.where(iota == r, val, resv)

    res[...] = resv
    off = pl.multiple_of(b * L, 8)
    pltpu.sync_copy(res, out_hbm.at[pl.ds(off, L)])


def kernel(x):
    tail = lax.slice(x, (0, 0, NT * TB), (B, H, N))
    xt = jnp.pad(tail, ((0, 0), (0, 0), (0, TB - TAILC)),
                 constant_values=-jnp.inf)
    out = _argmax_sc(x, xt)
    return out.reshape(B, L)[:, :H]


# speedup vs baseline: 1.1958x; 1.0234x over previous
import functools

import jax
import jax.numpy as jnp
from jax import lax
from jax.experimental import pallas as pl
from jax.experimental.pallas import tpu as pltpu
from jax.experimental.pallas import tpu_sc as plsc

B, H, N = 32, 8, 100000
L = 16
NC, NS = 2, 16
TB = 128
CTW = 13
WC = CTW * TB
RING = 4
NCH = 60
NROUND = NCH // RING
XTC = 2 * TB
GX = (NCH * WC) // L


@functools.partial(
    pl.kernel,
    mesh=plsc.VectorSubcoreMesh(core_axis_name="c", subcore_axis_name="s"),
    out_type=jax.ShapeDtypeStruct((B * L,), jnp.float32),
    scratch_types=[
        pltpu.VMEM((RING, H, WC), jnp.float32),
        pltpu.VMEM((H, XTC), jnp.float32),
        pltpu.VMEM((H, L), jnp.float32),
        pltpu.VMEM((H, L), jnp.int32),
        pltpu.VMEM((L,), jnp.float32),
        pltpu.SemaphoreType.DMA,
        pltpu.SemaphoreType.DMA,
        pltpu.SemaphoreType.DMA,
        pltpu.SemaphoreType.DMA,
        pltpu.SemaphoreType.DMA,
    ],
)
def _argmax_sc(x_hbm, xt_hbm, out_hbm, buf, tbuf, mbuf, abuf, res,
               sem0, sem1, sem2, sem3, semt):
    c = lax.axis_index("c")
    s = lax.axis_index("s")
    b = s * NC + c
    sems = (sem0, sem1, sem2, sem3)
    iota = lax.iota(jnp.int32, L)

    def chunk_copy(ci, slot):
        return pltpu.make_async_copy(
            x_hbm.at[b, :, pl.ds(ci * WC, WC)], buf.at[slot], sems[slot])

    for k in range(RING):
        chunk_copy(k, k).start()
    pltpu.make_async_copy(xt_hbm.at[b], tbuf, semt).start()

    def scan_groups(bufref, gbase, ngroups, ms, aas):
        def body(g, carry):
            mm = list(carry[:H])
            aa = list(carry[H:])
            col = g * L
            gv = jnp.broadcast_to(gbase + g, (L,))
            for r in range(H):
                v = bufref[r, pl.ds(col, L)]
                gt = v > mm[r]
                mm[r] = jnp.where(gt, v, mm[r])
                aa[r] = jnp.where(gt, gv, aa[r])
            return tuple(mm) + tuple(aa)

        carry = lax.fori_loop(0, ngroups, body, tuple(ms) + tuple(aas))
        return list(carry[:H]), list(carry[H:])

    ms = [jnp.full((L,), -jnp.inf, dtype=jnp.float32) for _ in range(H)]
    aas = [jnp.zeros((L,), dtype=jnp.int32) for _ in range(H)]

    def round_body(t, carry):
        ms = list(carry[:H])
        aas = list(carry[H:])
        for k in range(RING):
            ci = t * RING + k
            chunk_copy(ci, k).wait()
            ms, aas = scan_groups(buf.at[k], ci * (WC // L), WC // L, ms, aas)

            @pl.when(ci + RING < NCH)
            def _(ci=ci, k=k):
                chunk_copy(ci + RING, k).start()

        return tuple(ms) + tuple(aas)

    carry = lax.fori_loop(0, NROUND, round_body, tuple(ms) + tuple(aas))
    ms, aas = list(carry[:H]), list(carry[H:])

    pltpu.make_async_copy(xt_hbm.at[b], tbuf, semt).wait()
    ms, aas = scan_groups(tbuf, GX, XTC // L, ms, aas)

    for r in range(H):
        mbuf[r] = ms[r]
        abuf[r] = aas[r]

    def head_body(r, resv):
        rm = mbuf[r]
        ra = (abuf[r] << 4) + iota
        for sh in (8, 4, 2, 1):
            perm = iota ^ sh
            mo = rm.at[perm].get(mode="promise_in_bounds")
            ao = ra.at[perm].get(mode="promise_in_bounds")
            better = (mo > rm) | ((mo == rm) & (ao < ra))
            rm = jnp.where(better, mo, rm)
            ra = jnp.where(better, ao, ra)
        val = ra.astype(jnp.float32) / jnp.float32(N)
        return jnp.where(iota == r, val, resv)

    res[...] = lax.fori_loop(0, H, head_body,
                             jnp.zeros((L,), dtype=jnp.float32))
    off = pl.multiple_of(b * L, 8)
    pltpu.sync_copy(res, out_hbm.at[pl.ds(off, L)])


def kernel(x):
    tail = lax.slice(x, (0, 0, NCH * WC), (B, H, N))
    xt = jnp.pad(tail, ((0, 0), (0, 0), (0, XTC - (N - NCH * WC))),
                 constant_values=-jnp.inf)
    out = _argmax_sc(x, xt)
    return out.reshape(B, L)[:, :H]
